# baseline (speedup 1.0000x reference)
import functools

import jax
import jax.numpy as jnp
from jax import lax
from jax.experimental import pallas as pl
from jax.experimental.pallas import tpu as pltpu
from jax.experimental.pallas import tpu_sc as plsc

_NC = 2
_NS = 16
_NW = _NC * _NS
_CH = 128
_D = 64


def _sc_gather(uid2, iid2, ue, ie):
    B = uid2.shape[0] * _CH
    bpw = B // _NW
    nch = bpw // _CH
    mesh = plsc.VectorSubcoreMesh(core_axis_name="c", subcore_axis_name="s")

    @functools.partial(
        pl.kernel,
        mesh=mesh,
        out_type=[jax.ShapeDtypeStruct((B, 2 * _D), jnp.float32)] * 2,
        scratch_types=[
            pltpu.VMEM((nch, _CH), jnp.int32),
            pltpu.VMEM((nch, _CH), jnp.int32),
            pltpu.VMEM((_CH, 2 * _D), jnp.float32),
            pltpu.VMEM((_CH, 2 * _D), jnp.float32),
            pltpu.SemaphoreType.DMA,
            pltpu.SemaphoreType.DMA,
            pltpu.SemaphoreType.DMA,
            pltpu.SemaphoreType.DMA,
        ],
    )
    def k(uid_h, iid_h, ue_h, ie_h, o_u, o_i,
          gxu, gxi, buf0, buf1, g0, g1, w0, w1):
        wid = lax.axis_index("s") * _NC + lax.axis_index("c")
        base = wid * bpw
        pltpu.sync_copy(uid_h.at[pl.ds(wid * nch, nch)], gxu)
        pltpu.sync_copy(iid_h.at[pl.ds(wid * nch, nch)], gxi)
        specs = ((ue_h, gxu, o_u), (ie_h, gxi, o_i))
        buf = (buf0, buf1)
        gsem = (g0, g1)
        wsem = (w0, w1)
        ntot = 2 * nch

        def issue(n, b):
            t, c = divmod(n, nch)
            tab, gx, _ = specs[t]
            return pltpu.async_copy(tab.at[gx.at[c]], buf[b], gsem[b])

        pend = [issue(0, 0), None]
        wd = [None, None]
        for n in range(ntot):
            b = n % 2
            if n + 1 < ntot:
                if wd[1 - b] is not None:
                    wd[1 - b].wait()
                    wd[1 - b] = None
                pend[1 - b] = issue(n + 1, 1 - b)
            pend[b].wait()
            t, c = divmod(n, nch)
            out = specs[t][2]
            wd[b] = pltpu.async_copy(
                buf[b], out.at[pl.ds(base + c * _CH, _CH)], wsem[b])
        wd[0].wait()
        wd[1].wait()

    return k(uid2, iid2, ue, ie)


def _mlp_body(ur, ir, w1u, w1i, b1, w2, b2, w3, b3, wg, wh, bf, out):
    u = ur[...]
    iv = ir[...]
    h = jnp.dot(u[:, _D:], w1u[...], preferred_element_type=jnp.float32)
    h += jnp.dot(iv[:, _D:], w1i[...], preferred_element_type=jnp.float32)
    h = jnp.maximum(h + b1[...], 0.0)
    h = jnp.maximum(
        jnp.dot(h, w2[...], preferred_element_type=jnp.float32) + b2[...], 0.0)
    h = jnp.maximum(
        jnp.dot(h, w3[...], preferred_element_type=jnp.float32) + b3[...], 0.0)
    gmf = u[:, :_D] * iv[:, :_D]
    logit = (jnp.dot(gmf, wg[...], preferred_element_type=jnp.float32)
             + jnp.dot(h, wh[...], preferred_element_type=jnp.float32)
             + bf[0, 0])
    out[...] = 1.0 / (1.0 + jnp.exp(-logit))


def kernel(user_ids, item_ids, ue_gmf, ie_gmf, ue_mlp, ie_mlp,
           W1, b1, W2, b2, W3, b3, Wf, bf):
    B = user_ids.shape[0]
    D = ue_gmf.shape[1]
    ue = jnp.concatenate([ue_gmf, ue_mlp], axis=1)
    ie = jnp.concatenate([ie_gmf, ie_mlp], axis=1)
    uid2 = user_ids.reshape(B // _CH, _CH)
    iid2 = item_ids.reshape(B // _CH, _CH)
    u_rows, i_rows = _sc_gather(uid2, iid2, ue, ie)

    H1 = W1.shape[0]
    H2 = W2.shape[0]
    H3 = W3.shape[0]
    w1u = W1[:, :D].T
    w1i = W1[:, D:].T
    w2t = W2.T
    w3t = W3.T
    wg = Wf[:, :D].T
    wh = Wf[:, D:].T
    b1r = b1.reshape(1, H1)
    b2r = b2.reshape(1, H2)
    b3r = b3.reshape(1, H3)
    bfr = bf.reshape(1, 1)

    bB = 2048
    grid = (B // bB,)
    row_spec = pl.BlockSpec((bB, 2 * D), lambda i: (i, 0))

    def _w(shape):
        return pl.BlockSpec(shape, lambda i: (0, 0))

    out2 = pl.pallas_call(
        _mlp_body,
        grid=grid,
        in_specs=[
            row_spec, row_spec,
            _w((D, H1)), _w((D, H1)), _w((1, H1)),
            _w((H1, H2)), _w((1, H2)),
            _w((H2, H3)), _w((1, H3)),
            _w((D, 1)), _w((H3, 1)), _w((1, 1)),
        ],
        out_specs=pl.BlockSpec((bB, 1), lambda i: (i, 0)),
        out_shape=jax.ShapeDtypeStruct((B, 1), jnp.float32),
    )(u_rows, i_rows, w1u, w1i, b1r, w2t, b2r, w3t, b3r, wg, wh, bfr)
    return out2.reshape(B)
